# Initial kernel scaffold; baseline (speedup 1.0000x reference)
import jax
import jax.numpy as jnp
from jax.experimental import pallas as pl


def kernel(features, edge_index, W_sgc, W_lin, W_proj):
    raise NotImplementedError("write your pallas kernel here")


# speedup vs baseline: 6.9028x; 6.9028x over previous
import dataclasses
import functools

import jax
import jax.numpy as jnp
from jax import lax
from jax.experimental import pallas as pl
from jax.experimental.pallas import tpu as pltpu
from jax.experimental.pallas import tpu_sc as plsc

N = 10000
E = 320000
F = 128
H = 128
C = 64

NC = 2
NS = 16
NW = NC * NS

N_PAD = 10240
RPT = N_PAD // NS

CHUNK = 80
EPW = E // NW
NCHUNK = EPW // CHUNK
EPW_DEG = E // NS
NCHUNK_DEG = EPW_DEG // CHUNK

_MESH = plsc.VectorSubcoreMesh(
    core_axis_name="c", subcore_axis_name="s", num_cores=NC, num_subcores=NS
)

_SC_PARAMS = pltpu.CompilerParams()
if "needs_layout_passes" in pltpu.CompilerParams.__dataclass_fields__:
    _SC_PARAMS = dataclasses.replace(_SC_PARAMS, needs_layout_passes=False)


def _rsqrt_newton(x):
    i = plsc.bitcast(x, jnp.int32)
    i = jnp.int32(0x5F3759DF) - lax.shift_right_logical(i, 1)
    y = plsc.bitcast(i, jnp.float32)
    for _ in range(3):
        y = y * (1.5 - 0.5 * x * y * y)
    return y


@functools.partial(
    pl.kernel,
    out_type=jax.ShapeDtypeStruct((N_PAD,), jnp.float32),
    mesh=_MESH,
    scratch_types=[
        pltpu.VMEM((NCHUNK_DEG, CHUNK), jnp.int32),
        pltpu.VMEM((CHUNK,), jnp.float32),
        pltpu.VMEM((RPT,), jnp.float32),
        pltpu.VMEM_SHARED((N_PAD,), jnp.float32),
        pltpu.SemaphoreType.DMA,
    ],
    compiler_params=_SC_PARAMS,
)
def _deg_norm_kernel(dst_hbm, zeros_hbm, ones_hbm, norm_hbm,
                     dst_v, ones_v, slice_v, acc_sh, sem):
    c = lax.axis_index("c")
    s = lax.axis_index("s")

    @pl.when(c == 0)
    def _():
        base = s * RPT
        pltpu.sync_copy(zeros_hbm, acc_sh.at[pl.ds(base, RPT)])
        pltpu.sync_copy(dst_hbm.at[s], dst_v)
        pltpu.sync_copy(ones_hbm, ones_v)
        plsc.subcore_barrier()

        @pl.loop(0, NCHUNK_DEG)
        def _(j):
            pltpu.sync_copy(ones_v, acc_sh.at[dst_v.at[j]], add=True)

        plsc.subcore_barrier()

        pltpu.sync_copy(acc_sh.at[pl.ds(base, RPT)], slice_v)

        @pl.loop(0, RPT, step=16)
        def _(k):
            d = slice_v[pl.ds(k, 16)]
            slice_v[pl.ds(k, 16)] = jnp.where(d > 0.5, _rsqrt_newton(d), 0.0)

        pltpu.sync_copy(slice_v, norm_hbm.at[pl.ds(base, RPT)])


@functools.partial(
    pl.kernel,
    out_type=jax.ShapeDtypeStruct((NC, N_PAD, F), jnp.float32),
    mesh=_MESH,
    scratch_types=[
        pltpu.VMEM((NCHUNK, CHUNK), jnp.int32),
        pltpu.VMEM((NCHUNK, CHUNK), jnp.int32),
        pltpu.VMEM((CHUNK, F), jnp.float32),
        pltpu.VMEM_SHARED((N_PAD, F), jnp.float32),
        pltpu.SemaphoreType.DMA,
    ],
    compiler_params=_SC_PARAMS,
)
def _hop_kernel(g_hbm, src_hbm, dst_hbm, zeros_hbm, out_hbm,
                src_v, dst_v, rows_v, acc_sh, sem):
    c = lax.axis_index("c")
    s = lax.axis_index("s")
    w = c * NS + s
    base = s * RPT

    pltpu.sync_copy(zeros_hbm, acc_sh.at[pl.ds(base, RPT)])
    pltpu.sync_copy(src_hbm.at[w], src_v)
    pltpu.sync_copy(dst_hbm.at[w], dst_v)
    plsc.subcore_barrier()

    @pl.loop(0, NCHUNK)
    def _(j):
        pltpu.async_copy(g_hbm.at[src_v.at[j]], rows_v, sem).wait()
        pltpu.sync_copy(rows_v, acc_sh.at[dst_v.at[j]], add=True)

    plsc.subcore_barrier()
    pltpu.sync_copy(acc_sh.at[pl.ds(base, RPT)],
                    out_hbm.at[c, pl.ds(base, RPT)])


def _prep_body(feat_ref, norm_ref, out_ref):
    out_ref[...] = feat_ref[...] * norm_ref[...]


def _mid_body(p_ref, norm_ref, out_ref):
    n2 = norm_ref[...] * norm_ref[...]
    out_ref[...] = (p_ref[0, :N, :] + p_ref[1, :N, :]) * n2


def _dot_t(a, b):
    return lax.dot_general(a, b, (((1,), (1,)), ((), ())),
                           precision=lax.Precision.HIGHEST,
                           preferred_element_type=jnp.float32)


def _final_body(p_ref, norm_ref, feat_ref, wsgc_ref, wlin_ref,
                wp1_ref, wp2_ref, out_ref):
    s2 = (p_ref[0, :N, :] + p_ref[1, :N, :]) * norm_ref[...]
    x2 = _dot_t(s2, wsgc_ref[...])
    x1 = _dot_t(feat_ref[...], wlin_ref[...])
    out_ref[...] = _dot_t(x1, wp1_ref[...]) + _dot_t(x2, wp2_ref[...])


_prep = pl.pallas_call(
    _prep_body, out_shape=jax.ShapeDtypeStruct((N, F), jnp.float32))
_mid = pl.pallas_call(
    _mid_body, out_shape=jax.ShapeDtypeStruct((N, F), jnp.float32))
_final = pl.pallas_call(
    _final_body, out_shape=jax.ShapeDtypeStruct((N, C), jnp.float32))


@jax.jit
def kernel(features, edge_index, W_sgc, W_lin, W_proj):
    src = edge_index[0].astype(jnp.int32)
    dst = edge_index[1].astype(jnp.int32)
    src32 = src.reshape(NW, NCHUNK, CHUNK)
    dst32 = dst.reshape(NW, NCHUNK, CHUNK)
    dst16 = dst.reshape(NS, NCHUNK_DEG, CHUNK)

    zeros_deg = jnp.zeros((RPT,), jnp.float32)
    ones_chunk = jnp.ones((CHUNK,), jnp.float32)
    zeros_rows = jnp.zeros((RPT, F), jnp.float32)

    norm_flat = _deg_norm_kernel(dst16, zeros_deg, ones_chunk)
    norm1 = norm_flat[:N].reshape(N, 1)

    g1 = _prep(features, norm1)
    p1 = _hop_kernel(g1, src32, dst32, zeros_rows)
    g2 = _mid(p1, norm1)
    p2 = _hop_kernel(g2, src32, dst32, zeros_rows)

    wp1 = W_proj[:, :H]
    wp2 = W_proj[:, H:]
    return _final(p2, norm1, features, W_sgc, W_lin, wp1, wp2)


# speedup vs baseline: 7.1290x; 1.0328x over previous
import dataclasses
import functools

import jax
import jax.numpy as jnp
from jax import lax
from jax.experimental import pallas as pl
from jax.experimental.pallas import tpu as pltpu
from jax.experimental.pallas import tpu_sc as plsc

N = 10000
E = 320000
F = 128
H = 128
C = 64
FH = F // 2

NC = 2
NS = 16
NW = NC * NS

N_PAD = 10240
RPT = N_PAD // NS

CHUNK = 40
EPW = E // NW
NCHUNK = EPW // CHUNK
EPW_DEG = E // NS
NCHUNK_DEG = EPW_DEG // CHUNK

_MESH = plsc.VectorSubcoreMesh(
    core_axis_name="c", subcore_axis_name="s", num_cores=NC, num_subcores=NS
)

_SC_PARAMS = pltpu.CompilerParams()
if "needs_layout_passes" in pltpu.CompilerParams.__dataclass_fields__:
    _SC_PARAMS = dataclasses.replace(_SC_PARAMS, needs_layout_passes=False)
if "use_tc_tiling_on_sc" in pltpu.CompilerParams.__dataclass_fields__:
    _SC_PARAMS = dataclasses.replace(_SC_PARAMS, use_tc_tiling_on_sc=False)


def _rsqrt_newton(x):
    i = plsc.bitcast(x, jnp.int32)
    i = jnp.int32(0x5F3759DF) - lax.shift_right_logical(i, 1)
    y = plsc.bitcast(i, jnp.float32)
    for _ in range(3):
        y = y * (1.5 - 0.5 * x * y * y)
    return y


@functools.partial(
    pl.kernel,
    out_type=jax.ShapeDtypeStruct((N_PAD,), jnp.float32),
    mesh=_MESH,
    scratch_types=[
        pltpu.VMEM((NCHUNK_DEG, CHUNK), jnp.int32),
        pltpu.VMEM((CHUNK,), jnp.float32),
        pltpu.VMEM((RPT,), jnp.float32),
        pltpu.VMEM_SHARED((N_PAD,), jnp.float32),
        pltpu.SemaphoreType.DMA,
    ],
    compiler_params=_SC_PARAMS,
)
def _deg_norm_kernel(dst_hbm, zeros_hbm, ones_hbm, norm_hbm,
                     dst_v, ones_v, slice_v, acc_sh, sem):
    c = lax.axis_index("c")
    s = lax.axis_index("s")

    @pl.when(c == 0)
    def _():
        base = s * RPT
        pltpu.sync_copy(zeros_hbm, acc_sh.at[pl.ds(base, RPT)])
        pltpu.sync_copy(dst_hbm.at[s], dst_v)
        pltpu.sync_copy(ones_hbm, ones_v)
        plsc.subcore_barrier()

        @pl.loop(0, NCHUNK_DEG)
        def _(j):
            pltpu.sync_copy(ones_v, acc_sh.at[dst_v.at[j]], add=True)

        plsc.subcore_barrier()

        pltpu.sync_copy(acc_sh.at[pl.ds(base, RPT)], slice_v)

        @pl.loop(0, RPT, step=16)
        def _(k):
            d = slice_v[pl.ds(k, 16)]
            slice_v[pl.ds(k, 16)] = jnp.where(d > 0.5, _rsqrt_newton(d), 0.0)

        pltpu.sync_copy(slice_v, norm_hbm.at[pl.ds(base, RPT)])


RING = 5
PRE = 3
_MAIN_END = ((NCHUNK - PRE) // RING) * RING

@functools.partial(
    pl.kernel,
    out_type=jax.ShapeDtypeStruct((2, NC, N_PAD, FH), jnp.float32),
    mesh=_MESH,
    scratch_types=[
        pltpu.VMEM((NCHUNK, CHUNK), jnp.int32),
        pltpu.VMEM((NCHUNK, CHUNK), jnp.int32),
        [pltpu.VMEM((CHUNK, FH), jnp.float32)] * RING,
        pltpu.VMEM_SHARED((N_PAD, FH), jnp.float32),
        [pltpu.SemaphoreType.DMA] * RING,
        [pltpu.SemaphoreType.DMA] * RING,
    ],
    compiler_params=_SC_PARAMS,
)
def _hop_kernel(g_lo_hbm, g_hi_hbm, src_hbm, dst_hbm, zeros_hbm, out_hbm,
                src_v, dst_v, bufs, acc_sh, gsems, ssems):
    c = lax.axis_index("c")
    s = lax.axis_index("s")
    w = c * NS + s
    base = s * RPT

    pltpu.sync_copy(src_hbm.at[w], src_v)
    pltpu.sync_copy(dst_hbm.at[w], dst_v)

    for h, g_hbm in ((0, g_lo_hbm), (1, g_hi_hbm)):
        def fire_gather(k, b):
            pltpu.async_copy(g_hbm.at[src_v.at[k]], bufs[b], gsems[b])

        def wait_gather(k, b):
            pltpu.make_async_copy(g_hbm.at[src_v.at[k]], bufs[b],
                                  gsems[b]).wait()

        def fire_scatter(k, b):
            pltpu.async_copy(bufs[b], acc_sh.at[dst_v.at[k]], ssems[b],
                             add=True)

        def wait_scatter(k, b):
            pltpu.make_async_copy(bufs[b], acc_sh.at[dst_v.at[k]],
                                  ssems[b]).wait()

        for k in range(PRE):
            fire_gather(k, k % RING)
        pltpu.sync_copy(zeros_hbm, acc_sh.at[pl.ds(base, RPT)])
        plsc.subcore_barrier()

        def step(k):
            wait_gather(k, k % RING)
            fire_scatter(k, k % RING)
            p = k + PRE
            if p < NCHUNK:
                q = p - RING
                if q >= 0:
                    wait_scatter(q, p % RING)
                fire_gather(p, p % RING)

        for k in range(RING):
            step(k)

        @pl.loop(RING, _MAIN_END, step=RING)
        def _(j):
            for b in range(RING):
                k = j + b
                wait_gather(k, b)
                fire_scatter(k, b)
                wait_scatter(k + PRE - RING, (b + PRE) % RING)
                fire_gather(k + PRE, (b + PRE) % RING)

        for k in range(_MAIN_END, NCHUNK):
            step(k)
        for k in range(NCHUNK - RING, NCHUNK):
            wait_scatter(k, k % RING)

        plsc.subcore_barrier()
        pltpu.sync_copy(acc_sh.at[pl.ds(base, RPT)],
                        out_hbm.at[h, c, pl.ds(base, RPT)])
        if h == 0:
            plsc.subcore_barrier()


BR = 1000
NBLK = N // BR


def _prep_body(flo_ref, fhi_ref, norm_ref, olo_ref, ohi_ref):
    olo_ref[...] = flo_ref[...] * norm_ref[...]
    ohi_ref[...] = fhi_ref[...] * norm_ref[...]


def _mid_body(p_ref, norm_ref, olo_ref, ohi_ref):
    n2 = norm_ref[...] * norm_ref[...]
    olo_ref[...] = (p_ref[0, 0] + p_ref[0, 1]) * n2
    ohi_ref[...] = (p_ref[1, 0] + p_ref[1, 1]) * n2


def _dot_t(a, b):
    return lax.dot_general(a, b, (((1,), (1,)), ((), ())),
                           precision=lax.Precision.HIGHEST,
                           preferred_element_type=jnp.float32)


def _final_body(p_ref, norm_ref, feat_ref, wsgc_lo_ref, wsgc_hi_ref,
                wlin_ref, wp1_ref, wp2_ref, out_ref):
    s2_lo = (p_ref[0, 0] + p_ref[0, 1]) * norm_ref[...]
    s2_hi = (p_ref[1, 0] + p_ref[1, 1]) * norm_ref[...]
    x2 = _dot_t(s2_lo, wsgc_lo_ref[...]) + _dot_t(s2_hi, wsgc_hi_ref[...])
    x1 = _dot_t(feat_ref[...], wlin_ref[...])
    out_ref[...] = _dot_t(x1, wp1_ref[...]) + _dot_t(x2, wp2_ref[...])


_half = jax.ShapeDtypeStruct((N, FH), jnp.float32)
_half_spec = pl.BlockSpec((BR, FH), lambda i: (i, 0))
_norm_spec = pl.BlockSpec((BR, 1), lambda i: (i, 0))
_p_spec = pl.BlockSpec((2, NC, BR, FH), lambda i: (0, 0, i, 0))
_feat_spec = pl.BlockSpec((BR, F), lambda i: (i, 0))


def _w_spec(r, c_):
    return pl.BlockSpec((r, c_), lambda i: (0, 0))


_prep = pl.pallas_call(
    _prep_body, out_shape=[_half, _half], grid=(NBLK,),
    in_specs=[_half_spec, _half_spec, _norm_spec],
    out_specs=[_half_spec, _half_spec])
_mid = pl.pallas_call(
    _mid_body, out_shape=[_half, _half], grid=(NBLK,),
    in_specs=[_p_spec, _norm_spec],
    out_specs=[_half_spec, _half_spec])
_final = pl.pallas_call(
    _final_body, out_shape=jax.ShapeDtypeStruct((N, C), jnp.float32),
    grid=(NBLK,),
    in_specs=[_p_spec, _norm_spec, _feat_spec, _w_spec(H, FH), _w_spec(H, FH),
              _w_spec(H, F), _w_spec(C, H), _w_spec(C, H)],
    out_specs=pl.BlockSpec((BR, C), lambda i: (i, 0)))


@jax.jit
def kernel(features, edge_index, W_sgc, W_lin, W_proj):
    src = edge_index[0].astype(jnp.int32)
    dst = edge_index[1].astype(jnp.int32)
    src32 = src.reshape(NW, NCHUNK, CHUNK)
    dst32 = dst.reshape(NW, NCHUNK, CHUNK)
    dst16 = dst.reshape(NS, NCHUNK_DEG, CHUNK)

    zeros_deg = jnp.zeros((RPT,), jnp.float32)
    ones_chunk = jnp.ones((CHUNK,), jnp.float32)
    zeros_rows = jnp.zeros((RPT, FH), jnp.float32)

    norm_flat = _deg_norm_kernel(dst16, zeros_deg, ones_chunk)
    norm1 = norm_flat[:N].reshape(N, 1)

    g1_lo, g1_hi = _prep(features[:, :FH], features[:, FH:], norm1)
    p1 = _hop_kernel(g1_lo, g1_hi, src32, dst32, zeros_rows)
    g2_lo, g2_hi = _mid(p1, norm1)
    p2 = _hop_kernel(g2_lo, g2_hi, src32, dst32, zeros_rows)

    return _final(p2, norm1, features, W_sgc[:, :FH], W_sgc[:, FH:],
                  W_lin, W_proj[:, :H], W_proj[:, H:])


# speedup vs baseline: 8.0753x; 1.1327x over previous
import dataclasses
import functools

import jax
import jax.numpy as jnp
from jax import lax
from jax.experimental import pallas as pl
from jax.experimental.pallas import tpu as pltpu
from jax.experimental.pallas import tpu_sc as plsc

N = 10000
E = 320000
F = 128
H = 128
C = 64
FH = F // 2

NC = 2
NS = 16
NW = NC * NS

N_PAD = 10240
RPT = N_PAD // NS

CHUNK = 80
EPW = E // NW
NCHUNK = EPW // CHUNK
EPW_DEG = E // NS
NCHUNK_DEG = EPW_DEG // CHUNK

_MESH = plsc.VectorSubcoreMesh(
    core_axis_name="c", subcore_axis_name="s", num_cores=NC, num_subcores=NS
)

_SC_PARAMS = pltpu.CompilerParams()
if "needs_layout_passes" in pltpu.CompilerParams.__dataclass_fields__:
    _SC_PARAMS = dataclasses.replace(_SC_PARAMS, needs_layout_passes=False)
if "use_tc_tiling_on_sc" in pltpu.CompilerParams.__dataclass_fields__:
    _SC_PARAMS = dataclasses.replace(_SC_PARAMS, use_tc_tiling_on_sc=False)


def _rsqrt_newton(x):
    i = plsc.bitcast(x, jnp.int32)
    i = jnp.int32(0x5F3759DF) - lax.shift_right_logical(i, 1)
    y = plsc.bitcast(i, jnp.float32)
    for _ in range(3):
        y = y * (1.5 - 0.5 * x * y * y)
    return y


@functools.partial(
    pl.kernel,
    out_type=jax.ShapeDtypeStruct((N_PAD,), jnp.float32),
    mesh=_MESH,
    scratch_types=[
        pltpu.VMEM((NCHUNK_DEG, CHUNK), jnp.int32),
        pltpu.VMEM((CHUNK,), jnp.float32),
        pltpu.VMEM((RPT,), jnp.float32),
        pltpu.VMEM_SHARED((N_PAD,), jnp.float32),
        pltpu.SemaphoreType.DMA,
    ],
    compiler_params=_SC_PARAMS,
)
def _deg_norm_kernel(dst_hbm, zeros_hbm, ones_hbm, norm_hbm,
                     dst_v, ones_v, slice_v, acc_sh, sem):
    c = lax.axis_index("c")
    s = lax.axis_index("s")

    @pl.when(c == 0)
    def _():
        base = s * RPT
        pltpu.sync_copy(zeros_hbm, acc_sh.at[pl.ds(base, RPT)])
        pltpu.sync_copy(dst_hbm.at[s], dst_v)
        pltpu.sync_copy(ones_hbm, ones_v)
        plsc.subcore_barrier()

        @pl.loop(0, NCHUNK_DEG, step=10)
        def _(j):
            for t in range(10):
                pltpu.async_copy(ones_v, acc_sh.at[dst_v.at[j + t]], sem,
                                 add=True)
            for t in range(10):
                pltpu.make_async_copy(ones_v, acc_sh.at[dst_v.at[j + t]],
                                      sem).wait()

        plsc.subcore_barrier()

        pltpu.sync_copy(acc_sh.at[pl.ds(base, RPT)], slice_v)

        @pl.loop(0, RPT, step=16)
        def _(k):
            d = slice_v[pl.ds(k, 16)]
            slice_v[pl.ds(k, 16)] = jnp.where(d > 0.5, _rsqrt_newton(d), 0.0)

        pltpu.sync_copy(slice_v, norm_hbm.at[pl.ds(base, RPT)])


RING = 4
PRE = 2
_MAIN_END = ((NCHUNK - PRE) // RING) * RING

@functools.partial(
    pl.kernel,
    out_type=jax.ShapeDtypeStruct((2, NC, N_PAD, FH), jnp.float32),
    mesh=_MESH,
    scratch_types=[
        pltpu.VMEM((NCHUNK, CHUNK), jnp.int32),
        pltpu.VMEM((NCHUNK, CHUNK), jnp.int32),
        [pltpu.VMEM((CHUNK, FH), jnp.float32)] * RING,
        pltpu.VMEM_SHARED((N_PAD, FH), jnp.float32),
        [pltpu.SemaphoreType.DMA] * RING,
        [pltpu.SemaphoreType.DMA] * RING,
    ],
    compiler_params=_SC_PARAMS,
)
def _hop_kernel(g_lo_hbm, g_hi_hbm, src_hbm, dst_hbm, zeros_hbm, out_hbm,
                src_v, dst_v, bufs, acc_sh, gsems, ssems):
    c = lax.axis_index("c")
    s = lax.axis_index("s")
    w = c * NS + s
    base = s * RPT

    pltpu.sync_copy(src_hbm.at[w], src_v)
    pltpu.sync_copy(dst_hbm.at[w], dst_v)

    for h, g_hbm in ((0, g_lo_hbm), (1, g_hi_hbm)):
        def fire_gather(k, b):
            pltpu.async_copy(g_hbm.at[src_v.at[k]], bufs[b], gsems[b])

        def wait_gather(k, b):
            pltpu.make_async_copy(g_hbm.at[src_v.at[k]], bufs[b],
                                  gsems[b]).wait()

        def fire_scatter(k, b):
            pltpu.async_copy(bufs[b], acc_sh.at[dst_v.at[k]], ssems[b],
                             add=True)

        def wait_scatter(k, b):
            pltpu.make_async_copy(bufs[b], acc_sh.at[dst_v.at[k]],
                                  ssems[b]).wait()

        for k in range(PRE):
            fire_gather(k, k % RING)
        pltpu.sync_copy(zeros_hbm, acc_sh.at[pl.ds(base, RPT)])
        plsc.subcore_barrier()

        def step(k):
            wait_gather(k, k % RING)
            fire_scatter(k, k % RING)
            p = k + PRE
            if p < NCHUNK:
                q = p - RING
                if q >= 0:
                    wait_scatter(q, p % RING)
                fire_gather(p, p % RING)

        for k in range(RING):
            step(k)

        @pl.loop(RING, _MAIN_END, step=RING)
        def _(j):
            for b in range(RING):
                k = j + b
                wait_gather(k, b)
                fire_scatter(k, b)
                wait_scatter(k + PRE - RING, (b + PRE) % RING)
                fire_gather(k + PRE, (b + PRE) % RING)

        for k in range(_MAIN_END, NCHUNK):
            step(k)
        for k in range(NCHUNK - RING, NCHUNK):
            wait_scatter(k, k % RING)

        plsc.subcore_barrier()
        pltpu.sync_copy(acc_sh.at[pl.ds(base, RPT)],
                        out_hbm.at[h, c, pl.ds(base, RPT)])
        if h == 0:
            plsc.subcore_barrier()


BR = 1000
NBLK = N // BR


def _prep_body(flo_ref, fhi_ref, norm_ref, olo_ref, ohi_ref):
    olo_ref[...] = flo_ref[...] * norm_ref[...]
    ohi_ref[...] = fhi_ref[...] * norm_ref[...]


def _mid_body(p_ref, norm_ref, olo_ref, ohi_ref):
    n2 = norm_ref[...] * norm_ref[...]
    olo_ref[...] = (p_ref[0, 0] + p_ref[0, 1]) * n2
    ohi_ref[...] = (p_ref[1, 0] + p_ref[1, 1]) * n2


def _dot_t(a, b):
    return lax.dot_general(a, b, (((1,), (1,)), ((), ())),
                           precision=lax.Precision.HIGHEST,
                           preferred_element_type=jnp.float32)


def _root_body(feat_ref, wlin_ref, wp1_ref, out_ref):
    x1 = _dot_t(feat_ref[...], wlin_ref[...])
    out_ref[...] = _dot_t(x1, wp1_ref[...])


def _final_body(p_ref, norm_ref, x1p_ref, wsgc_lo_ref, wsgc_hi_ref,
                wp2_ref, out_ref):
    s2_lo = (p_ref[0, 0] + p_ref[0, 1]) * norm_ref[...]
    s2_hi = (p_ref[1, 0] + p_ref[1, 1]) * norm_ref[...]
    x2 = _dot_t(s2_lo, wsgc_lo_ref[...]) + _dot_t(s2_hi, wsgc_hi_ref[...])
    out_ref[...] = x1p_ref[...] + _dot_t(x2, wp2_ref[...])


_half = jax.ShapeDtypeStruct((N, FH), jnp.float32)
_half_spec = pl.BlockSpec((BR, FH), lambda i: (i, 0))
_norm_spec = pl.BlockSpec((BR, 1), lambda i: (i, 0))
_p_spec = pl.BlockSpec((2, NC, BR, FH), lambda i: (0, 0, i, 0))
_feat_spec = pl.BlockSpec((BR, F), lambda i: (i, 0))


def _w_spec(r, c_):
    return pl.BlockSpec((r, c_), lambda i: (0, 0))


_prep = pl.pallas_call(
    _prep_body, out_shape=[_half, _half], grid=(NBLK,),
    in_specs=[_half_spec, _half_spec, _norm_spec],
    out_specs=[_half_spec, _half_spec])
_mid = pl.pallas_call(
    _mid_body, out_shape=[_half, _half], grid=(NBLK,),
    in_specs=[_p_spec, _norm_spec],
    out_specs=[_half_spec, _half_spec])
_out_spec = pl.BlockSpec((BR, C), lambda i: (i, 0))
_root = pl.pallas_call(
    _root_body, out_shape=jax.ShapeDtypeStruct((N, C), jnp.float32),
    grid=(NBLK,),
    in_specs=[_feat_spec, _w_spec(H, F), _w_spec(C, H)],
    out_specs=_out_spec)
_final = pl.pallas_call(
    _final_body, out_shape=jax.ShapeDtypeStruct((N, C), jnp.float32),
    grid=(NBLK,),
    in_specs=[_p_spec, _norm_spec, _out_spec, _w_spec(H, FH), _w_spec(H, FH),
              _w_spec(C, H)],
    out_specs=_out_spec)


@jax.jit
def kernel(features, edge_index, W_sgc, W_lin, W_proj):
    src = edge_index[0].astype(jnp.int32)
    dst = edge_index[1].astype(jnp.int32)
    src32 = src.reshape(NW, NCHUNK, CHUNK)
    dst32 = dst.reshape(NW, NCHUNK, CHUNK)
    dst16 = dst.reshape(NS, NCHUNK_DEG, CHUNK)

    zeros_deg = jnp.zeros((RPT,), jnp.float32)
    ones_chunk = jnp.ones((CHUNK,), jnp.float32)
    zeros_rows = jnp.zeros((RPT, FH), jnp.float32)

    x1p = _root(features, W_lin, W_proj[:, :H])

    norm_flat = _deg_norm_kernel(dst16, zeros_deg, ones_chunk)
    norm1 = norm_flat[:N].reshape(N, 1)

    g1_lo, g1_hi = _prep(features[:, :FH], features[:, FH:], norm1)
    p1 = _hop_kernel(g1_lo, g1_hi, src32, dst32, zeros_rows)
    g2_lo, g2_hi = _mid(p1, norm1)
    p2 = _hop_kernel(g2_lo, g2_hi, src32, dst32, zeros_rows)

    return _final(p2, norm1, x1p, W_sgc[:, :FH], W_sgc[:, FH:],
                  W_proj[:, H:])


# speedup vs baseline: 8.4996x; 1.0525x over previous
import dataclasses
import functools

import jax
import jax.numpy as jnp
from jax import lax
from jax.experimental import pallas as pl
from jax.experimental.pallas import tpu as pltpu
from jax.experimental.pallas import tpu_sc as plsc

N = 10000
E = 320000
F = 128
H = 128
C = 64
FH = F // 2

NC = 2
NS = 16
NW = NC * NS

N_PAD = 10240
RPT = N_PAD // NS

CHUNK = 80
EPW = E // NW
NCHUNK = EPW // CHUNK
EPW_DEG = E // NS
NCHUNK_DEG = EPW_DEG // CHUNK

_MESH = plsc.VectorSubcoreMesh(
    core_axis_name="c", subcore_axis_name="s", num_cores=NC, num_subcores=NS
)

_SC_PARAMS = pltpu.CompilerParams()
if "needs_layout_passes" in pltpu.CompilerParams.__dataclass_fields__:
    _SC_PARAMS = dataclasses.replace(_SC_PARAMS, needs_layout_passes=False)
if "use_tc_tiling_on_sc" in pltpu.CompilerParams.__dataclass_fields__:
    _SC_PARAMS = dataclasses.replace(_SC_PARAMS, use_tc_tiling_on_sc=False)


def _rsqrt_newton(x):
    i = plsc.bitcast(x, jnp.int32)
    i = jnp.int32(0x5F3759DF) - lax.shift_right_logical(i, 1)
    y = plsc.bitcast(i, jnp.float32)
    for _ in range(3):
        y = y * (1.5 - 0.5 * x * y * y)
    return y


@functools.partial(
    pl.kernel,
    out_type=jax.ShapeDtypeStruct((N_PAD,), jnp.float32),
    mesh=_MESH,
    scratch_types=[
        pltpu.VMEM((NCHUNK_DEG, CHUNK), jnp.int32),
        pltpu.VMEM((CHUNK,), jnp.float32),
        pltpu.VMEM((RPT,), jnp.float32),
        pltpu.VMEM_SHARED((N_PAD,), jnp.float32),
        pltpu.SemaphoreType.DMA,
    ],
    compiler_params=_SC_PARAMS,
)
def _deg_norm_kernel(dst_hbm, zeros_hbm, ones_hbm, norm_hbm,
                     dst_v, ones_v, slice_v, acc_sh, sem):
    c = lax.axis_index("c")
    s = lax.axis_index("s")

    @pl.when(c == 0)
    def _():
        base = s * RPT
        pltpu.sync_copy(zeros_hbm, acc_sh.at[pl.ds(base, RPT)])
        pltpu.sync_copy(dst_hbm.at[s], dst_v)
        pltpu.sync_copy(ones_hbm, ones_v)
        plsc.subcore_barrier()

        @pl.loop(0, NCHUNK_DEG, step=10)
        def _(j):
            for t in range(10):
                pltpu.async_copy(ones_v, acc_sh.at[dst_v.at[j + t]], sem,
                                 add=True)
            for t in range(10):
                pltpu.make_async_copy(ones_v, acc_sh.at[dst_v.at[j + t]],
                                      sem).wait()

        plsc.subcore_barrier()

        pltpu.sync_copy(acc_sh.at[pl.ds(base, RPT)], slice_v)

        @pl.loop(0, RPT, step=16)
        def _(k):
            d = slice_v[pl.ds(k, 16)]
            slice_v[pl.ds(k, 16)] = jnp.where(d > 0.5, _rsqrt_newton(d), 0.0)

        pltpu.sync_copy(slice_v, norm_hbm.at[pl.ds(base, RPT)])


RING = 4
PRE = 2
_MAIN_END = ((NCHUNK - PRE) // RING) * RING

@functools.partial(
    pl.kernel,
    out_type=jax.ShapeDtypeStruct((2, NC, N_PAD, FH), jnp.float32),
    mesh=_MESH,
    scratch_types=[
        pltpu.VMEM((NCHUNK, CHUNK), jnp.int32),
        pltpu.VMEM((NCHUNK, CHUNK), jnp.int32),
        [pltpu.VMEM((CHUNK, FH), jnp.float32)] * RING,
        pltpu.VMEM_SHARED((N_PAD, FH), jnp.float32),
        [pltpu.SemaphoreType.DMA] * RING,
        [pltpu.SemaphoreType.DMA] * RING,
    ],
    compiler_params=_SC_PARAMS,
)
def _hop_kernel(g_lo_hbm, g_hi_hbm, src_hbm, dst_hbm, zeros_hbm, out_hbm,
                src_v, dst_v, bufs, acc_sh, gsems, ssems):
    c = lax.axis_index("c")
    s = lax.axis_index("s")
    w = c * NS + s
    base = s * RPT

    pltpu.sync_copy(src_hbm.at[w], src_v)
    pltpu.sync_copy(dst_hbm.at[w], dst_v)

    for h, g_hbm in ((0, g_lo_hbm), (1, g_hi_hbm)):
        def fire_gather(k, b):
            pltpu.async_copy(g_hbm.at[src_v.at[k]], bufs[b], gsems[b])

        def wait_gather(k, b):
            pltpu.make_async_copy(g_hbm.at[src_v.at[k]], bufs[b],
                                  gsems[b]).wait()

        def fire_scatter(k, b):
            pltpu.async_copy(bufs[b], acc_sh.at[dst_v.at[k]], ssems[b],
                             add=True)

        def wait_scatter(k, b):
            pltpu.make_async_copy(bufs[b], acc_sh.at[dst_v.at[k]],
                                  ssems[b]).wait()

        for k in range(PRE):
            fire_gather(k, k % RING)
        pltpu.sync_copy(zeros_hbm, acc_sh.at[pl.ds(base, RPT)])
        plsc.subcore_barrier()

        def step(k):
            wait_gather(k, k % RING)
            fire_scatter(k, k % RING)
            p = k + PRE
            if p < NCHUNK:
                q = p - RING
                if q >= 0:
                    wait_scatter(q, p % RING)
                fire_gather(p, p % RING)

        for k in range(RING):
            step(k)

        @pl.loop(RING, _MAIN_END, step=RING)
        def _(j):
            for b in range(RING):
                k = j + b
                wait_gather(k, b)
                fire_scatter(k, b)
                wait_scatter(k + PRE - RING, (b + PRE) % RING)
                fire_gather(k + PRE, (b + PRE) % RING)

        for k in range(_MAIN_END, NCHUNK):
            step(k)
        for k in range(NCHUNK - RING, NCHUNK):
            wait_scatter(k, k % RING)

        plsc.subcore_barrier()
        pltpu.sync_copy(acc_sh.at[pl.ds(base, RPT)],
                        out_hbm.at[h, c, pl.ds(base, RPT)])
        if h == 0:
            plsc.subcore_barrier()


BR = 1000
NBLK = N // BR


def _prep_body(flo_ref, fhi_ref, norm_ref, olo_ref, ohi_ref):
    olo_ref[...] = flo_ref[...] * norm_ref[...]
    ohi_ref[...] = fhi_ref[...] * norm_ref[...]


def _mid_body(p_ref, norm_ref, olo_ref, ohi_ref):
    n2 = norm_ref[...] * norm_ref[...]
    olo_ref[...] = (p_ref[0, 0] + p_ref[0, 1]) * n2
    ohi_ref[...] = (p_ref[1, 0] + p_ref[1, 1]) * n2


def _dot_t(a, b):
    return lax.dot_general(a, b, (((1,), (1,)), ((), ())),
                           precision=lax.Precision.HIGHEST,
                           preferred_element_type=jnp.float32)


def _fold_body(wsgc_ref, wlin_ref, wproj_ref, w1_ref, w2_ref):
    dn = (((1,), (0,)), ((), ()))
    w1_ref[...] = lax.dot_general(
        wproj_ref[:, :H], wlin_ref[...], dn,
        precision=lax.Precision.HIGHEST, preferred_element_type=jnp.float32)
    w2_ref[...] = lax.dot_general(
        wproj_ref[:, H:], wsgc_ref[...], dn,
        precision=lax.Precision.HIGHEST, preferred_element_type=jnp.float32)


def _root_body(feat_ref, w1_ref, out_ref):
    out_ref[...] = _dot_t(feat_ref[...], w1_ref[...])


def _final_body(p_ref, norm_ref, x1p_ref, w2_ref, out_ref):
    s2_lo = (p_ref[0, 0] + p_ref[0, 1]) * norm_ref[...]
    s2_hi = (p_ref[1, 0] + p_ref[1, 1]) * norm_ref[...]
    out_ref[...] = (x1p_ref[...] + _dot_t(s2_lo, w2_ref[:, :FH])
                    + _dot_t(s2_hi, w2_ref[:, FH:]))


_half = jax.ShapeDtypeStruct((N, FH), jnp.float32)
_half_spec = pl.BlockSpec((BR, FH), lambda i: (i, 0))
_norm_spec = pl.BlockSpec((BR, 1), lambda i: (i, 0))
_p_spec = pl.BlockSpec((2, NC, BR, FH), lambda i: (0, 0, i, 0))
_feat_spec = pl.BlockSpec((BR, F), lambda i: (i, 0))


def _w_spec(r, c_):
    return pl.BlockSpec((r, c_), lambda i: (0, 0))


_prep = pl.pallas_call(
    _prep_body, out_shape=[_half, _half], grid=(NBLK,),
    in_specs=[_half_spec, _half_spec, _norm_spec],
    out_specs=[_half_spec, _half_spec])
_mid = pl.pallas_call(
    _mid_body, out_shape=[_half, _half], grid=(NBLK,),
    in_specs=[_p_spec, _norm_spec],
    out_specs=[_half_spec, _half_spec])
_out_spec = pl.BlockSpec((BR, C), lambda i: (i, 0))
_wcf = jax.ShapeDtypeStruct((C, F), jnp.float32)
_fold = pl.pallas_call(_fold_body, out_shape=[_wcf, _wcf])
_root = pl.pallas_call(
    _root_body, out_shape=jax.ShapeDtypeStruct((N, C), jnp.float32),
    grid=(NBLK,),
    in_specs=[_feat_spec, _w_spec(C, F)],
    out_specs=_out_spec)
_final = pl.pallas_call(
    _final_body, out_shape=jax.ShapeDtypeStruct((N, C), jnp.float32),
    grid=(NBLK,),
    in_specs=[_p_spec, _norm_spec, _out_spec, _w_spec(C, F)],
    out_specs=_out_spec)


@jax.jit
def kernel(features, edge_index, W_sgc, W_lin, W_proj):
    src = edge_index[0].astype(jnp.int32)
    dst = edge_index[1].astype(jnp.int32)
    src32 = src.reshape(NW, NCHUNK, CHUNK)
    dst32 = dst.reshape(NW, NCHUNK, CHUNK)
    dst16 = dst.reshape(NS, NCHUNK_DEG, CHUNK)

    zeros_deg = jnp.zeros((RPT,), jnp.float32)
    ones_chunk = jnp.ones((CHUNK,), jnp.float32)
    zeros_rows = jnp.zeros((RPT, FH), jnp.float32)

    w1, w2 = _fold(W_sgc, W_lin, W_proj)
    x1p = _root(features, w1)

    norm_flat = _deg_norm_kernel(dst16, zeros_deg, ones_chunk)
    norm1 = norm_flat[:N].reshape(N, 1)

    g1_lo, g1_hi = _prep(features[:, :FH], features[:, FH:], norm1)
    p1 = _hop_kernel(g1_lo, g1_hi, src32, dst32, zeros_rows)
    g2_lo, g2_hi = _mid(p1, norm1)
    p2 = _hop_kernel(g2_lo, g2_hi, src32, dst32, zeros_rows)

    return _final(p2, norm1, x1p, w2)
